# baseline (speedup 1.0000x reference)
import functools

import jax
import jax.numpy as jnp
from jax import lax
from jax.experimental import pallas as pl
from jax.experimental.pallas import tpu as pltpu
from jax.experimental.pallas import tpu_sc as plsc


def _make_sc_kernel(batch, n_factors):
    info = plsc.get_sparse_core_info()
    nc, ns, lanes = info.num_cores, info.num_subcores, info.num_lanes
    nw = nc * ns
    assert batch % (8 * nw) == 0
    bpw = batch // nw
    half = bpw // 2
    npairs = n_factors // 2
    roww = 4 * npairs
    mesh = plsc.VectorSubcoreMesh(core_axis_name="c", subcore_axis_name="s")

    @functools.partial(
        pl.kernel,
        out_type=jax.ShapeDtypeStruct((batch,), jnp.float32),
        mesh=mesh,
        compiler_params=pltpu.CompilerParams(
            needs_layout_passes=False, use_tc_tiling_on_sc=False
        ),
        scratch_types=[
            pltpu.VMEM((bpw,), jnp.int32),
            pltpu.VMEM((bpw,), jnp.int32),
            pltpu.VMEM((bpw,), jnp.int32),
            pltpu.VMEM((bpw,), jnp.int32),
            pltpu.VMEM((bpw,), jnp.int32),
            pltpu.VMEM((bpw,), jnp.int32),
            pltpu.VMEM((half, roww), jnp.int32),
            pltpu.VMEM((half, roww), jnp.int32),
            pltpu.VMEM((bpw,), jnp.float32),
            pltpu.VMEM((bpw,), jnp.float32),
            pltpu.VMEM((bpw,), jnp.float32),
            pltpu.SemaphoreType.DMA,
            pltpu.SemaphoreType.DMA,
        ],
    )
    def mf_kernel(urow_hbm, mrow_hbm, ucol_hbm, mcol_hbm, uorig_hbm, morig_hbm,
                  uf_hbm, mf_hbm, ub_hbm, mb_hbm, out_hbm, uridx, mridx,
                  ucol, mcol, uorig, morig, urows, mrows, ubias, mbias, outv,
                  sem, bsem):
        wid = lax.axis_index("s") * nc + lax.axis_index("c")
        base = wid * bpw
        pltpu.sync_copy(urow_hbm.at[pl.ds(base, bpw)], uridx)
        pltpu.sync_copy(mrow_hbm.at[pl.ds(base, bpw)], mridx)
        pltpu.sync_copy(ucol_hbm.at[pl.ds(base, bpw)], ucol)
        pltpu.sync_copy(mcol_hbm.at[pl.ds(base, bpw)], mcol)
        pltpu.sync_copy(uorig_hbm.at[pl.ds(base, bpw)], uorig)
        pltpu.sync_copy(morig_hbm.at[pl.ds(base, bpw)], morig)
        cb1 = pltpu.async_copy(ub_hbm.at[uorig], ubias, bsem)
        cb2 = pltpu.async_copy(mb_hbm.at[morig], mbias, bsem)

        for h in range(2):
            hb = h * half
            c1 = pltpu.async_copy(uf_hbm.at[uridx.at[pl.ds(hb, half)]], urows, sem)
            c2 = pltpu.async_copy(mf_hbm.at[mridx.at[pl.ds(hb, half)]], mrows, sem)
            c1.wait()
            c2.wait()

            def group(g, _):
                rows = g * lanes + lax.iota(jnp.int32, lanes)
                ucols = ucol[pl.ds(hb + g * lanes, lanes)]
                mcols = mcol[pl.ds(hb + g * lanes, lanes)]
                acc = jnp.zeros((lanes,), jnp.float32)
                for p in range(npairs):
                    uw = plsc.load_gather(urows, [rows, ucols + p])
                    mw = plsc.load_gather(mrows, [rows, mcols + p])
                    ua, ub_ = plsc.unpack(plsc.bitcast(uw, jnp.bfloat16),
                                          format=plsc.PackFormat.INTERLEAVED)
                    ma, mb_ = plsc.unpack(plsc.bitcast(mw, jnp.bfloat16),
                                          format=plsc.PackFormat.INTERLEAVED)
                    acc = acc + ua * ma + ub_ * mb_
                outv[pl.ds(hb + g * lanes, lanes)] = acc
                return 0

            lax.fori_loop(0, half // lanes, group, 0)

        cb1.wait()
        cb2.wait()

        def addb(g, _):
            sl = pl.ds(g * lanes, lanes)
            outv[sl] = outv[sl] + ubias[sl] + mbias[sl]
            return 0

        lax.fori_loop(0, bpw // lanes, addb, 0)
        pltpu.sync_copy(outv, out_hbm.at[pl.ds(base, bpw)])

    return mf_kernel


def _pack_table(table):
    n, d = table.shape
    bf = table.astype(jnp.bfloat16)
    as_i32 = jax.lax.bitcast_convert_type(bf.reshape(n, d // 2, 2), jnp.int32)
    return as_i32.reshape(n // 4, 2 * d)


def kernel(user, movie, user_factors, movie_factors, user_biases, movie_biases):
    batch = user.shape[0]
    n_factors = user_factors.shape[1]
    mf_kernel = _make_sc_kernel(batch, n_factors)
    user = user.astype(jnp.int32)
    movie = movie.astype(jnp.int32)
    return mf_kernel(
        user // 4,
        movie // 4,
        (user % 4) * (n_factors // 2),
        (movie % 4) * (n_factors // 2),
        user,
        movie,
        _pack_table(user_factors),
        _pack_table(movie_factors),
        user_biases.reshape(-1),
        movie_biases.reshape(-1),
    )


# speedup vs baseline: 2.1439x; 2.1439x over previous
import functools

import jax
import jax.numpy as jnp
from jax import lax
from jax.experimental import pallas as pl
from jax.experimental.pallas import tpu as pltpu
from jax.experimental.pallas import tpu_sc as plsc


def _make_sc_kernel(batch, n_factors):
    info = plsc.get_sparse_core_info()
    nc, ns, lanes = info.num_cores, info.num_subcores, info.num_lanes
    nw = nc * ns
    assert batch % (8 * nw) == 0
    bpw = batch // nw
    half = bpw // 2
    mesh = plsc.VectorSubcoreMesh(core_axis_name="c", subcore_axis_name="s")

    @functools.partial(
        pl.kernel,
        out_type=jax.ShapeDtypeStruct((batch,), jnp.float32),
        mesh=mesh,
        compiler_params=pltpu.CompilerParams(
            needs_layout_passes=False,
            use_tc_tiling_on_sc=False,
            skip_device_barrier=True,
        ),
        scratch_types=[
            pltpu.VMEM((bpw,), jnp.int32),
            pltpu.VMEM((bpw,), jnp.int32),
            pltpu.VMEM((bpw,), jnp.int32),
            pltpu.VMEM((bpw,), jnp.int32),
            pltpu.VMEM((bpw,), jnp.int32),
            pltpu.VMEM((bpw,), jnp.int32),
            pltpu.VMEM((half, 128), jnp.float32),
            pltpu.VMEM((half, 128), jnp.float32),
            pltpu.VMEM((bpw,), jnp.float32),
            pltpu.VMEM((bpw,), jnp.float32),
            pltpu.VMEM((bpw,), jnp.float32),
            pltpu.SemaphoreType.DMA,
            pltpu.SemaphoreType.DMA,
        ],
    )
    def mf_kernel(urow_hbm, mrow_hbm, ucol_hbm, mcol_hbm, uorig_hbm, morig_hbm,
                  uf_hbm, mf_hbm, ub_hbm, mb_hbm, out_hbm, uridx, mridx,
                  ucol, mcol, uorig, morig, urows, mrows, ubias, mbias, outv,
                  sem, bsem):
        wid = lax.axis_index("s") * nc + lax.axis_index("c")
        base = wid * bpw
        pltpu.sync_copy(urow_hbm.at[pl.ds(base, bpw)], uridx)
        pltpu.sync_copy(mrow_hbm.at[pl.ds(base, bpw)], mridx)
        pltpu.sync_copy(ucol_hbm.at[pl.ds(base, bpw)], ucol)
        pltpu.sync_copy(mcol_hbm.at[pl.ds(base, bpw)], mcol)
        pltpu.sync_copy(uorig_hbm.at[pl.ds(base, bpw)], uorig)
        pltpu.sync_copy(morig_hbm.at[pl.ds(base, bpw)], morig)
        cb1 = pltpu.async_copy(ub_hbm.at[uorig], ubias, bsem)
        cb2 = pltpu.async_copy(mb_hbm.at[morig], mbias, bsem)

        for h in range(2):
            hb = h * half
            c1 = pltpu.async_copy(uf_hbm.at[uridx.at[pl.ds(hb, half)]], urows, sem)
            c2 = pltpu.async_copy(mf_hbm.at[mridx.at[pl.ds(hb, half)]], mrows, sem)
            c1.wait()
            c2.wait()

            def group(g, _):
                rows = g * lanes + lax.iota(jnp.int32, lanes)
                ucols = ucol[pl.ds(hb + g * lanes, lanes)]
                mcols = mcol[pl.ds(hb + g * lanes, lanes)]
                acc = jnp.zeros((lanes,), jnp.float32)
                for f in range(n_factors):
                    uv = plsc.load_gather(urows, [rows, ucols + f])
                    mv = plsc.load_gather(mrows, [rows, mcols + f])
                    acc = acc + uv * mv
                outv[pl.ds(hb + g * lanes, lanes)] = acc
                return 0

            lax.fori_loop(0, half // lanes, group, 0)

        cb1.wait()
        cb2.wait()

        def addb(g, _):
            sl = pl.ds(g * lanes, lanes)
            outv[sl] = outv[sl] + ubias[sl] + mbias[sl]
            return 0

        lax.fori_loop(0, bpw // lanes, addb, 0)
        pltpu.sync_copy(outv, out_hbm.at[pl.ds(base, bpw)])

    return mf_kernel


def kernel(user, movie, user_factors, movie_factors, user_biases, movie_biases):
    batch = user.shape[0]
    n_factors = user_factors.shape[1]
    pack = 128 // n_factors
    mf_kernel = _make_sc_kernel(batch, n_factors)
    user = user.astype(jnp.int32)
    movie = movie.astype(jnp.int32)
    n_users = user_factors.shape[0]
    n_movies = movie_factors.shape[0]
    return mf_kernel(
        user // pack,
        movie // pack,
        (user % pack) * n_factors,
        (movie % pack) * n_factors,
        user,
        movie,
        user_factors.reshape(n_users // pack, pack * n_factors),
        movie_factors.reshape(n_movies // pack, pack * n_factors),
        user_biases.reshape(-1),
        movie_biases.reshape(-1),
    )


# speedup vs baseline: 2.1785x; 1.0161x over previous
import functools

import jax
import jax.numpy as jnp
from jax import lax
from jax.experimental import pallas as pl
from jax.experimental.pallas import tpu as pltpu
from jax.experimental.pallas import tpu_sc as plsc


def _make_sc_kernel(batch, n_factors):
    info = plsc.get_sparse_core_info()
    nc, ns, lanes = info.num_cores, info.num_subcores, info.num_lanes
    nw = nc * ns
    assert batch % (8 * nw) == 0
    bpw = batch // nw
    mesh = plsc.VectorSubcoreMesh(core_axis_name="c", subcore_axis_name="s")

    @functools.partial(
        pl.kernel,
        out_type=jax.ShapeDtypeStruct((batch,), jnp.float32),
        mesh=mesh,
        compiler_params=pltpu.CompilerParams(
            needs_layout_passes=False, use_tc_tiling_on_sc=False
        ),
        scratch_types=[
            pltpu.VMEM((bpw,), jnp.int32),
            pltpu.VMEM((bpw,), jnp.int32),
            pltpu.VMEM((bpw, n_factors), jnp.float32),
            pltpu.VMEM((bpw, n_factors), jnp.float32),
            pltpu.VMEM((bpw,), jnp.float32),
            pltpu.VMEM((bpw,), jnp.float32),
            pltpu.VMEM((bpw,), jnp.float32),
            pltpu.SemaphoreType.DMA,
        ],
    )
    def mf_kernel(user_hbm, movie_hbm, uf_hbm, mf_hbm, ub_hbm, mb_hbm,
                  out_hbm, uidx, midx, urows, mrows, ubias, mbias, outv, sem):
        wid = lax.axis_index("s") * nc + lax.axis_index("c")
        base = wid * bpw
        pltpu.sync_copy(user_hbm.at[pl.ds(base, bpw)], uidx)
        pltpu.sync_copy(movie_hbm.at[pl.ds(base, bpw)], midx)
        c1 = pltpu.async_copy(uf_hbm.at[uidx], urows, sem)
        c2 = pltpu.async_copy(mf_hbm.at[midx], mrows, sem)
        c3 = pltpu.async_copy(ub_hbm.at[uidx], ubias, sem)
        c4 = pltpu.async_copy(mb_hbm.at[midx], mbias, sem)
        c1.wait()
        c2.wait()
        c3.wait()
        c4.wait()

        def group(g, _):
            rows = g * lanes + lax.iota(jnp.int32, lanes)
            acc = ubias[pl.ds(g * lanes, lanes)] + mbias[pl.ds(g * lanes, lanes)]
            for f in range(n_factors):
                cols = jnp.full((lanes,), f, jnp.int32)
                uv = plsc.load_gather(urows, [rows, cols])
                mv = plsc.load_gather(mrows, [rows, cols])
                acc = acc + uv * mv
            outv[pl.ds(g * lanes, lanes)] = acc
            return 0

        lax.fori_loop(0, bpw // lanes, group, 0)
        pltpu.sync_copy(outv, out_hbm.at[pl.ds(base, bpw)])

    return mf_kernel


def kernel(user, movie, user_factors, movie_factors, user_biases, movie_biases):
    batch = user.shape[0]
    n_factors = user_factors.shape[1]
    mf_kernel = _make_sc_kernel(batch, n_factors)
    scale = (user[0] * 0 + 1).astype(jnp.float32)
    return mf_kernel(
        user.astype(jnp.int32),
        movie.astype(jnp.int32),
        user_factors * scale,
        movie_factors * scale,
        user_biases.reshape(-1),
        movie_biases.reshape(-1),
    )
